# baseline (speedup 1.0000x reference)
import functools

import jax
import jax.numpy as jnp
from jax import lax
from jax.experimental import pallas as pl
from jax.experimental.pallas import tpu as pltpu
from jax.experimental.pallas import tpu_sc as plsc

NC = 2
NS = 16
NW = NC * NS
CHUNK = 128
ROW_BLK = 256


def _make_agg(npad, d, ept):
  chunks = ept // CHUNK
  assert chunks % 2 == 0
  rpt = npad // NS
  mesh = plsc.VectorSubcoreMesh(core_axis_name="c", subcore_axis_name="s")

  @functools.partial(
      pl.kernel,
      mesh=mesh,
      out_type=jax.ShapeDtypeStruct((NC, npad, d), jnp.float32),
      scratch_types=[
          pltpu.VMEM((2, CHUNK), jnp.int32),
          pltpu.VMEM((CHUNK, d), jnp.float32),
          pltpu.VMEM_SHARED((npad, d), jnp.float32),
          pltpu.SemaphoreType.DMA,
      ],
  )
  def agg(h_hbm, edges_hbm, zrows_hbm, out_hbm, eidx, rows, acc, gsem):
    c = lax.axis_index("c")
    s = lax.axis_index("s")
    wid = s * NC + c
    pltpu.sync_copy(zrows_hbm, acc.at[pl.ds(s * rpt, rpt)])
    plsc.subcore_barrier()

    def body(j, carry):
      pltpu.sync_copy(edges_hbm.at[wid, j], eidx)
      pltpu.async_copy(h_hbm.at[eidx.at[0]], rows, gsem).wait()
      pltpu.sync_copy(rows, acc.at[eidx.at[1]], add=True)
      return carry

    lax.fori_loop(0, chunks, body, 0)
    plsc.subcore_barrier()
    pltpu.sync_copy(acc.at[pl.ds(s * rpt, rpt)],
                    out_hbm.at[c, pl.ds(s * rpt, rpt)])

  return agg


def _make_deg(npad, ept):
  chunks = ept // CHUNK
  rpt = npad // NS
  mesh = plsc.VectorSubcoreMesh(core_axis_name="c", subcore_axis_name="s")

  @functools.partial(
      pl.kernel,
      mesh=mesh,
      out_type=jax.ShapeDtypeStruct((NC, npad, 128), jnp.float32),
      scratch_types=[
          pltpu.VMEM((chunks, 2, CHUNK), jnp.int32),
          pltpu.VMEM((CHUNK, 128), jnp.float32),
          pltpu.VMEM_SHARED((npad, 128), jnp.float32),
      ],
  )
  def deg(edges_hbm, zrows_hbm, ones_hbm, out_hbm, didx, ones_v, acc):
    c = lax.axis_index("c")
    s = lax.axis_index("s")
    wid = s * NC + c
    pltpu.sync_copy(zrows_hbm, acc.at[pl.ds(s * rpt, rpt)])
    pltpu.sync_copy(ones_hbm, ones_v)
    pltpu.sync_copy(edges_hbm.at[wid], didx)
    plsc.subcore_barrier()

    def body(j, carry):
      pltpu.sync_copy(ones_v, acc.at[didx.at[j, 1]], add=True)
      return carry

    lax.fori_loop(0, chunks, body, 0)
    plsc.subcore_barrier()
    pltpu.sync_copy(acc.at[pl.ds(s * rpt, rpt)],
                    out_hbm.at[c, pl.ds(s * rpt, rpt)])

  return deg


def _sage_mm_body(h_ref, a0_ref, a1_ref, d0_ref, d1_ref, ws_ref, wn_ref,
                  b_ref, o_ref, *, relu):
  deg = d0_ref[0, :, 0:1] + d1_ref[0, :, 0:1]
  r = 1.0 / jnp.maximum(deg, 1.0)
  hn = (a0_ref[0] + a1_ref[0]) * r
  acc = jnp.dot(h_ref[...], ws_ref[...], preferred_element_type=jnp.float32)
  acc = acc + jnp.dot(hn, wn_ref[...], preferred_element_type=jnp.float32)
  acc = acc + b_ref[...]
  if relu:
    acc = jnp.maximum(acc, 0.0)
  o_ref[...] = acc


def _sage_mm(h, aggp, degp, ws, wn, b, relu):
  npad, din = h.shape
  dout = ws.shape[1]
  grid = npad // ROW_BLK
  return pl.pallas_call(
      functools.partial(_sage_mm_body, relu=relu),
      grid=(grid,),
      in_specs=[
          pl.BlockSpec((ROW_BLK, din), lambda i: (i, 0)),
          pl.BlockSpec((1, ROW_BLK, din), lambda i: (0, i, 0)),
          pl.BlockSpec((1, ROW_BLK, din), lambda i: (1, i, 0)),
          pl.BlockSpec((1, ROW_BLK, 128), lambda i: (0, i, 0)),
          pl.BlockSpec((1, ROW_BLK, 128), lambda i: (1, i, 0)),
          pl.BlockSpec((din, dout), lambda i: (0, 0)),
          pl.BlockSpec((din, dout), lambda i: (0, 0)),
          pl.BlockSpec((1, dout), lambda i: (0, 0)),
      ],
      out_specs=pl.BlockSpec((ROW_BLK, dout), lambda i: (i, 0)),
      out_shape=jax.ShapeDtypeStruct((npad, dout), jnp.float32),
  )(h, aggp, aggp, degp, degp, ws, wn, b)


def _proj_body(h_ref, ws_ref, wn_ref, b_ref, s_ref, p_ref):
  h = h_ref[...]
  s_ref[...] = (
      jnp.dot(h, ws_ref[...], preferred_element_type=jnp.float32) + b_ref[...])
  p_ref[...] = jnp.dot(h, wn_ref[...], preferred_element_type=jnp.float32)


def _proj(h, ws, wn, b):
  npad, din = h.shape
  dout = ws.shape[1]
  grid = npad // ROW_BLK
  return pl.pallas_call(
      _proj_body,
      grid=(grid,),
      in_specs=[
          pl.BlockSpec((ROW_BLK, din), lambda i: (i, 0)),
          pl.BlockSpec((din, dout), lambda i: (0, 0)),
          pl.BlockSpec((din, dout), lambda i: (0, 0)),
          pl.BlockSpec((1, dout), lambda i: (0, 0)),
      ],
      out_specs=[
          pl.BlockSpec((ROW_BLK, dout), lambda i: (i, 0)),
          pl.BlockSpec((ROW_BLK, dout), lambda i: (i, 0)),
      ],
      out_shape=[
          jax.ShapeDtypeStruct((npad, dout), jnp.float32),
          jax.ShapeDtypeStruct((npad, dout), jnp.float32),
      ],
  )(h, ws, wn, b)


def _final_body(s_ref, a0_ref, a1_ref, d0_ref, d1_ref, o_ref):
  deg = d0_ref[0, :, 0:1] + d1_ref[0, :, 0:1]
  r = 1.0 / jnp.maximum(deg, 1.0)
  o_ref[...] = s_ref[...] + (a0_ref[0] + a1_ref[0]) * r


def _final(s, aggp, degp):
  npad, dout = s.shape
  grid = npad // ROW_BLK
  return pl.pallas_call(
      _final_body,
      grid=(grid,),
      in_specs=[
          pl.BlockSpec((ROW_BLK, dout), lambda i: (i, 0)),
          pl.BlockSpec((1, ROW_BLK, dout), lambda i: (0, i, 0)),
          pl.BlockSpec((1, ROW_BLK, dout), lambda i: (1, i, 0)),
          pl.BlockSpec((1, ROW_BLK, 128), lambda i: (0, i, 0)),
          pl.BlockSpec((1, ROW_BLK, 128), lambda i: (1, i, 0)),
      ],
      out_specs=pl.BlockSpec((ROW_BLK, dout), lambda i: (i, 0)),
      out_shape=jax.ShapeDtypeStruct((npad, dout), jnp.float32),
  )(s, aggp, aggp, degp, degp)


def kernel(x, edge_index, W_self_0, W_neigh_0, b_0, W_self_1, W_neigh_1, b_1,
           W_self_2, W_neigh_2, b_2):
  n, d_in = x.shape
  d_h = W_self_0.shape[1]
  d_out = W_self_2.shape[1]
  e = edge_index.shape[1]

  lcm = ROW_BLK
  while lcm % (NS * 8) != 0:
    lcm += ROW_BLK
  npad = -(-(n + 1) // lcm) * lcm
  ept = -(-e // (NW * CHUNK)) * CHUNK
  if (ept // CHUNK) % 2:
    ept += CHUNK
  epad = NW * ept - e
  chunks = ept // CHUNK
  src = jnp.concatenate([edge_index[0], jnp.zeros((epad,), jnp.int32)])
  dst = jnp.concatenate(
      [edge_index[1], jnp.full((epad,), n, dtype=jnp.int32)])
  edges = jnp.stack(
      [src.reshape(NW, chunks, CHUNK), dst.reshape(NW, chunks, CHUNK)],
      axis=2)

  rpt = npad // NS
  z_h = jnp.zeros((rpt, d_h), jnp.float32)
  z_128 = jnp.zeros((rpt, 128), jnp.float32)
  ones128 = jnp.ones((CHUNK, 128), jnp.float32)

  x_p = jnp.pad(x, ((0, npad - n), (0, 0)))
  b0 = b_0.reshape(1, -1)
  b1 = b_1.reshape(1, -1)
  b2 = b_2.reshape(1, -1)

  deg_fn = _make_deg(npad, ept)
  agg_x = _make_agg(npad, d_in, ept)
  agg_h = _make_agg(npad, d_h, ept)

  degp = deg_fn(edges, z_128, ones128)

  a = agg_x(x_p, edges, jnp.zeros((rpt, d_in), jnp.float32))
  h1 = _sage_mm(x_p, a, degp, W_self_0, W_neigh_0, b0, relu=True)

  a = agg_h(h1, edges, z_h)
  h2 = _sage_mm(h1, a, degp, W_self_1, W_neigh_1, b1, relu=True)

  a = agg_h(h2, edges, z_h)
  out = _sage_mm(h2, a, degp, W_self_2, W_neigh_2, b2, relu=False)
  return out[:n]


# speedup vs baseline: 1.2813x; 1.2813x over previous
import functools

import jax
import jax.numpy as jnp
from jax import lax
from jax.experimental import pallas as pl
from jax.experimental.pallas import tpu as pltpu
from jax.experimental.pallas import tpu_sc as plsc

NC = 2
NS = 16
NW = NC * NS
CHUNK = 128
ROW_BLK = 256


def _make_agg(npad, d, ept):
  chunks = ept // CHUNK
  rpt = npad // NS
  mesh = plsc.VectorSubcoreMesh(core_axis_name="c", subcore_axis_name="s")

  @functools.partial(
      pl.kernel,
      mesh=mesh,
      out_type=jax.ShapeDtypeStruct((NC, npad, d), jnp.float32),
      scratch_types=[
          pltpu.VMEM((CHUNK,), jnp.int32),
          pltpu.VMEM((CHUNK,), jnp.int32),
          pltpu.VMEM((CHUNK, d), jnp.float32),
          pltpu.VMEM_SHARED((npad, d), jnp.float32),
          pltpu.SemaphoreType.DMA,
      ],
  )
  def agg(h_hbm, src_hbm, dst_hbm, zrows_hbm, out_hbm, sidx, didx, rows, acc,
          sem):
    c = lax.axis_index("c")
    s = lax.axis_index("s")
    wid = s * NC + c
    pltpu.sync_copy(zrows_hbm, acc.at[pl.ds(s * rpt, rpt)])
    plsc.subcore_barrier()

    base0 = wid * ept

    def body(j, carry):
      base = pl.multiple_of(base0 + j * CHUNK, CHUNK)
      pltpu.sync_copy(src_hbm.at[pl.ds(base, CHUNK)], sidx)
      pltpu.sync_copy(dst_hbm.at[pl.ds(base, CHUNK)], didx)
      pltpu.async_copy(h_hbm.at[sidx], rows, sem).wait()
      pltpu.sync_copy(rows, acc.at[didx], add=True)
      return carry

    lax.fori_loop(0, chunks, body, 0)
    plsc.subcore_barrier()
    pltpu.sync_copy(acc.at[pl.ds(s * rpt, rpt)],
                    out_hbm.at[c, pl.ds(s * rpt, rpt)])

  return agg


def _make_deg(npad, ept):
  chunks = ept // CHUNK
  rpt = npad // NS
  mesh = plsc.VectorSubcoreMesh(core_axis_name="c", subcore_axis_name="s")

  @functools.partial(
      pl.kernel,
      mesh=mesh,
      out_type=jax.ShapeDtypeStruct((NC, npad, 128), jnp.float32),
      scratch_types=[
          pltpu.VMEM((CHUNK,), jnp.int32),
          pltpu.VMEM((CHUNK, 128), jnp.float32),
          pltpu.VMEM_SHARED((npad, 128), jnp.float32),
      ],
  )
  def deg(dst_hbm, zrows_hbm, ones_hbm, out_hbm, didx, ones_v, acc):
    c = lax.axis_index("c")
    s = lax.axis_index("s")
    wid = s * NC + c
    pltpu.sync_copy(zrows_hbm, acc.at[pl.ds(s * rpt, rpt)])
    pltpu.sync_copy(ones_hbm, ones_v)
    plsc.subcore_barrier()

    base0 = wid * ept

    def body(j, carry):
      base = pl.multiple_of(base0 + j * CHUNK, CHUNK)
      pltpu.sync_copy(dst_hbm.at[pl.ds(base, CHUNK)], didx)
      pltpu.sync_copy(ones_v, acc.at[didx], add=True)
      return carry

    lax.fori_loop(0, chunks, body, 0)
    plsc.subcore_barrier()
    pltpu.sync_copy(acc.at[pl.ds(s * rpt, rpt)],
                    out_hbm.at[c, pl.ds(s * rpt, rpt)])

  return deg


def _sage_mm_body(h_ref, a0_ref, a1_ref, d0_ref, d1_ref, ws_ref, wn_ref,
                  b_ref, o_ref, *, relu):
  deg = d0_ref[0, :, 0:1] + d1_ref[0, :, 0:1]
  r = 1.0 / jnp.maximum(deg, 1.0)
  hn = (a0_ref[0] + a1_ref[0]) * r
  acc = jnp.dot(h_ref[...], ws_ref[...], preferred_element_type=jnp.float32)
  acc = acc + jnp.dot(hn, wn_ref[...], preferred_element_type=jnp.float32)
  acc = acc + b_ref[...]
  if relu:
    acc = jnp.maximum(acc, 0.0)
  o_ref[...] = acc


def _sage_mm(h, aggp, degp, ws, wn, b, relu):
  npad, din = h.shape
  dout = ws.shape[1]
  grid = npad // ROW_BLK
  return pl.pallas_call(
      functools.partial(_sage_mm_body, relu=relu),
      grid=(grid,),
      in_specs=[
          pl.BlockSpec((ROW_BLK, din), lambda i: (i, 0)),
          pl.BlockSpec((1, ROW_BLK, din), lambda i: (0, i, 0)),
          pl.BlockSpec((1, ROW_BLK, din), lambda i: (1, i, 0)),
          pl.BlockSpec((1, ROW_BLK, 128), lambda i: (0, i, 0)),
          pl.BlockSpec((1, ROW_BLK, 128), lambda i: (1, i, 0)),
          pl.BlockSpec((din, dout), lambda i: (0, 0)),
          pl.BlockSpec((din, dout), lambda i: (0, 0)),
          pl.BlockSpec((1, dout), lambda i: (0, 0)),
      ],
      out_specs=pl.BlockSpec((ROW_BLK, dout), lambda i: (i, 0)),
      out_shape=jax.ShapeDtypeStruct((npad, dout), jnp.float32),
  )(h, aggp, aggp, degp, degp, ws, wn, b)


def _proj_body(h_ref, ws_ref, wn_ref, b_ref, s_ref, p_ref):
  h = h_ref[...]
  s_ref[...] = (
      jnp.dot(h, ws_ref[...], preferred_element_type=jnp.float32) + b_ref[...])
  p_ref[...] = jnp.dot(h, wn_ref[...], preferred_element_type=jnp.float32)


def _proj(h, ws, wn, b):
  npad, din = h.shape
  dout = ws.shape[1]
  grid = npad // ROW_BLK
  return pl.pallas_call(
      _proj_body,
      grid=(grid,),
      in_specs=[
          pl.BlockSpec((ROW_BLK, din), lambda i: (i, 0)),
          pl.BlockSpec((din, dout), lambda i: (0, 0)),
          pl.BlockSpec((din, dout), lambda i: (0, 0)),
          pl.BlockSpec((1, dout), lambda i: (0, 0)),
      ],
      out_specs=[
          pl.BlockSpec((ROW_BLK, dout), lambda i: (i, 0)),
          pl.BlockSpec((ROW_BLK, dout), lambda i: (i, 0)),
      ],
      out_shape=[
          jax.ShapeDtypeStruct((npad, dout), jnp.float32),
          jax.ShapeDtypeStruct((npad, dout), jnp.float32),
      ],
  )(h, ws, wn, b)


def _final_body(s_ref, a0_ref, a1_ref, d0_ref, d1_ref, o_ref):
  deg = d0_ref[0, :, 0:1] + d1_ref[0, :, 0:1]
  r = 1.0 / jnp.maximum(deg, 1.0)
  o_ref[...] = s_ref[...] + (a0_ref[0] + a1_ref[0]) * r


def _final(s, aggp, degp):
  npad, dout = s.shape
  grid = npad // ROW_BLK
  return pl.pallas_call(
      _final_body,
      grid=(grid,),
      in_specs=[
          pl.BlockSpec((ROW_BLK, dout), lambda i: (i, 0)),
          pl.BlockSpec((1, ROW_BLK, dout), lambda i: (0, i, 0)),
          pl.BlockSpec((1, ROW_BLK, dout), lambda i: (1, i, 0)),
          pl.BlockSpec((1, ROW_BLK, 128), lambda i: (0, i, 0)),
          pl.BlockSpec((1, ROW_BLK, 128), lambda i: (1, i, 0)),
      ],
      out_specs=pl.BlockSpec((ROW_BLK, dout), lambda i: (i, 0)),
      out_shape=jax.ShapeDtypeStruct((npad, dout), jnp.float32),
  )(s, aggp, aggp, degp, degp)


def kernel(x, edge_index, W_self_0, W_neigh_0, b_0, W_self_1, W_neigh_1, b_1,
           W_self_2, W_neigh_2, b_2):
  n, d_in = x.shape
  d_h = W_self_0.shape[1]
  d_out = W_self_2.shape[1]
  e = edge_index.shape[1]

  lcm = ROW_BLK
  while lcm % (NS * 8) != 0:
    lcm += ROW_BLK
  npad = -(-(n + 1) // lcm) * lcm
  ept = -(-e // (NW * CHUNK)) * CHUNK
  epad = NW * ept - e
  src = jnp.concatenate([edge_index[0], jnp.zeros((epad,), jnp.int32)])
  dst = jnp.concatenate(
      [edge_index[1], jnp.full((epad,), n, dtype=jnp.int32)])

  rpt = npad // NS
  z_h = jnp.zeros((rpt, d_h), jnp.float32)
  z_128 = jnp.zeros((rpt, 128), jnp.float32)
  ones128 = jnp.ones((CHUNK, 128), jnp.float32)

  x_p = jnp.pad(x, ((0, npad - n), (0, 0)))
  b0 = b_0.reshape(1, -1)
  b1 = b_1.reshape(1, -1)
  b2 = b_2.reshape(1, -1)

  deg_fn = _make_deg(npad, ept)
  agg_x = _make_agg(npad, d_in, ept)
  agg_h = _make_agg(npad, d_h, ept)

  degp = deg_fn(dst, z_128, ones128)

  a = agg_x(x_p, src, dst, jnp.zeros((rpt, d_in), jnp.float32))
  h1 = _sage_mm(x_p, a, degp, W_self_0, W_neigh_0, b0, relu=True)

  a = agg_h(h1, src, dst, z_h)
  h2 = _sage_mm(h1, a, degp, W_self_1, W_neigh_1, b1, relu=True)

  a = agg_h(h2, src, dst, z_h)
  out = _sage_mm(h2, a, degp, W_self_2, W_neigh_2, b2, relu=False)
  return out[:n]
